# baseline (speedup 1.0000x reference)
import functools

import jax
import jax.numpy as jnp
from jax import lax
from jax.experimental import pallas as pl
from jax.experimental.pallas import tpu as pltpu
from jax.experimental.pallas import tpu_sc as plsc

N = 10000
E = 320000
D = 128

NSPLIT = 2
ES = E // NSPLIT
NC = 2
NS = 16
NW = NC * NS
EPW = ES // NW
CHUNK = 40
NCHUNK = EPW // CHUNK
NBUF = 5
NOUTER = NCHUNK // NBUF
NP = 10240
RPS = NP // NS
RSTEP = CHUNK

_mesh = plsc.VectorSubcoreMesh(core_axis_name="c", subcore_axis_name="s")


def _proj_body(x_ref, w1_ref, b1_ref, xs_ref, xr_ref):
    x = x_ref[...]
    xs_ref[...] = (
        jnp.dot(x, w1_ref[0:D, :], preferred_element_type=jnp.float32)
        + b1_ref[...]
    )
    xr_ref[...] = jnp.dot(x, w1_ref[D:2 * D, :], preferred_element_type=jnp.float32)


def _edge_body(gs_ref, gr_ref, e_ref, w1e_ref, w2_ref, b2_ref, ne_ref, eo_ref):
    e = e_ref[...]
    pe = jnp.dot(e, w1e_ref[...], preferred_element_type=jnp.float32)
    h = jnp.maximum(gs_ref[...] + gr_ref[...] + pe, 0.0)
    tmp = jnp.dot(h, w2_ref[...], preferred_element_type=jnp.float32) + b2_ref[...]
    ne_ref[...] = tmp
    eo_ref[...] = tmp + e


def _edge_body_alias(gs_ref, gr_ref, e_ref, w1e_ref, w2_ref, b2_ref, eo_in_ref,
                     ne_ref, eo_ref):
    del eo_in_ref
    _edge_body(gs_ref, gr_ref, e_ref, w1e_ref, w2_ref, b2_ref, ne_ref, eo_ref)


def _node_body(x_ref, p0_ref, p1_ref, w1_ref, b1_ref, w2_ref, b2_ref, out_ref):
    x = x_ref[...]
    agg = (p0_ref[0] + p0_ref[1]) + (p1_ref[0] + p1_ref[1])
    h = jnp.maximum(
        jnp.dot(x, w1_ref[0:D, :], preferred_element_type=jnp.float32)
        + jnp.dot(agg, w1_ref[D:2 * D, :], preferred_element_type=jnp.float32)
        + b1_ref[...],
        0.0,
    )
    out_ref[...] = (
        jnp.dot(h, w2_ref[...], preferred_element_type=jnp.float32)
        + b2_ref[...]
        + x
    )


def _make_gather(soff):
    @functools.partial(
        pl.kernel,
        mesh=_mesh,
        out_type=[
            jax.ShapeDtypeStruct((ES, D), jnp.float32),
            jax.ShapeDtypeStruct((ES, D), jnp.float32),
        ],
        scratch_types=(
            [pltpu.VMEM((CHUNK,), jnp.int32) for _ in range(2 * NBUF)]
            + [pltpu.VMEM((CHUNK, D), jnp.float32) for _ in range(2 * NBUF)]
            + [pltpu.SemaphoreType.DMA for _ in range(3 * NBUF)]
        ),
    )
    def gather_sc(xs_hbm, xr_hbm, snd_hbm, rcv_hbm, gs_hbm, gr_hbm, *scr):
        idx_s = scr[0:NBUF]
        idx_r = scr[NBUF:2 * NBUF]
        rows_s = scr[2 * NBUF:3 * NBUF]
        rows_r = scr[3 * NBUF:4 * NBUF]
        sem_i = scr[4 * NBUF:5 * NBUF]
        sem_g = scr[5 * NBUF:6 * NBUF]
        sem_w = scr[6 * NBUF:7 * NBUF]

        wid = lax.axis_index("s") * NC + lax.axis_index("c")
        base = wid * EPW
        ibase = soff + base

        for b in range(NBUF):
            ioff = ibase + b * CHUNK
            pltpu.async_copy(snd_hbm.at[pl.ds(ioff, CHUNK)], idx_s[b], sem_i[b])
            pltpu.async_copy(rcv_hbm.at[pl.ds(ioff, CHUNK)], idx_r[b], sem_i[b])

        def outer(g, carry):
            for b in range(NBUF):
                k = g * NBUF + b
                off = base + k * CHUNK

                @pl.when(g > 0)
                def _drain():
                    pltpu.make_async_copy(
                        rows_s[b], gs_hbm.at[pl.ds(base, CHUNK)], sem_w[b]).wait()
                    pltpu.make_async_copy(
                        rows_r[b], gr_hbm.at[pl.ds(base, CHUNK)], sem_w[b]).wait()

                pltpu.make_async_copy(
                    snd_hbm.at[pl.ds(ibase, CHUNK)], idx_s[b], sem_i[b]).wait()
                pltpu.make_async_copy(
                    rcv_hbm.at[pl.ds(ibase, CHUNK)], idx_r[b], sem_i[b]).wait()
                cs = pltpu.async_copy(xs_hbm.at[idx_s[b]], rows_s[b], sem_g[b])
                cr = pltpu.async_copy(xr_hbm.at[idx_r[b]], rows_r[b], sem_g[b])
                cs.wait()
                cr.wait()
                pltpu.async_copy(rows_s[b], gs_hbm.at[pl.ds(off, CHUNK)], sem_w[b])
                pltpu.async_copy(rows_r[b], gr_hbm.at[pl.ds(off, CHUNK)], sem_w[b])

                @pl.when(k + NBUF < NCHUNK)
                def _prefetch():
                    ioff2 = ibase + (k + NBUF) * CHUNK
                    pltpu.async_copy(
                        snd_hbm.at[pl.ds(ioff2, CHUNK)], idx_s[b], sem_i[b])
                    pltpu.async_copy(
                        rcv_hbm.at[pl.ds(ioff2, CHUNK)], idx_r[b], sem_i[b])
            return carry

        lax.fori_loop(0, NOUTER, outer, 0)
        for b in range(NBUF):
            pltpu.make_async_copy(
                rows_s[b], gs_hbm.at[pl.ds(base, CHUNK)], sem_w[b]).wait()
            pltpu.make_async_copy(
                rows_r[b], gr_hbm.at[pl.ds(base, CHUNK)], sem_w[b]).wait()

    return gather_sc


def _make_scatter(soff):
    @functools.partial(
        pl.kernel,
        mesh=_mesh,
        out_type=jax.ShapeDtypeStruct((NC, NP, D), jnp.float32),
        scratch_types=(
            [pltpu.VMEM((CHUNK,), jnp.int32) for _ in range(NBUF)]
            + [pltpu.VMEM((CHUNK, D), jnp.float32) for _ in range(NBUF)]
            + [pltpu.VMEM_SHARED((NP, D), jnp.float32)]
            + [pltpu.SemaphoreType.DMA for _ in range(NBUF)]
        ),
    )
    def scatter_sc(ne_hbm, rcv_hbm, zero_hbm, parts_hbm, *scr):
        idx = scr[0:NBUF]
        chunk = scr[NBUF:2 * NBUF]
        acc = scr[2 * NBUF]
        sem_i = scr[2 * NBUF + 1:3 * NBUF + 1]

        cid = lax.axis_index("c")
        sid = lax.axis_index("s")
        wid = sid * NC + cid

        pltpu.sync_copy(zero_hbm.at[pl.ds(0, RSTEP)], chunk[0])

        def zbody(i, carry):
            r0 = sid * RPS + i * RSTEP
            pltpu.sync_copy(chunk[0], acc.at[pl.ds(r0, RSTEP)])
            return carry

        lax.fori_loop(0, RPS // RSTEP, zbody, 0)
        plsc.subcore_barrier()

        base = wid * EPW
        ibase = soff + base

        for b in range(NBUF):
            pltpu.async_copy(
                rcv_hbm.at[pl.ds(ibase + b * CHUNK, CHUNK)], idx[b], sem_i[b])
            pltpu.async_copy(
                ne_hbm.at[pl.ds(base + b * CHUNK, CHUNK)], chunk[b], sem_i[b])

        def outer(g, carry):
            for b in range(NBUF):
                k = g * NBUF + b
                pltpu.make_async_copy(
                    rcv_hbm.at[pl.ds(ibase, CHUNK)], idx[b], sem_i[b]).wait()
                pltpu.make_async_copy(
                    ne_hbm.at[pl.ds(base, CHUNK)], chunk[b], sem_i[b]).wait()
                pltpu.sync_copy(chunk[b], acc.at[idx[b]], add=True)

                @pl.when(k + NBUF < NCHUNK)
                def _prefetch():
                    k2 = k + NBUF
                    pltpu.async_copy(
                        rcv_hbm.at[pl.ds(ibase + k2 * CHUNK, CHUNK)],
                        idx[b], sem_i[b])
                    pltpu.async_copy(
                        ne_hbm.at[pl.ds(base + k2 * CHUNK, CHUNK)],
                        chunk[b], sem_i[b])
            return carry

        lax.fori_loop(0, NOUTER, outer, 0)
        plsc.subcore_barrier()

        def obody(i, carry):
            r0 = sid * RPS + i * RSTEP
            pltpu.sync_copy(acc.at[pl.ds(r0, RSTEP)], chunk[0])
            pltpu.sync_copy(chunk[0], parts_hbm.at[cid, pl.ds(r0, RSTEP)])
            return carry

        lax.fori_loop(0, RPS // RSTEP, obody, 0)

    return scatter_sc


_gather_calls = [_make_gather(s * ES) for s in range(NSPLIT)]
_scatter_calls = [_make_scatter(s * ES) for s in range(NSPLIT)]


def kernel(node_features, edge_features, me_w1, me_b1, me_w2, me_b2,
           nm_w1, nm_b1, nm_w2, nm_b2, senders, receivers):
    snd = senders.astype(jnp.int32)
    rcv = receivers.astype(jnp.int32)

    BN = 1000
    xs, xr = pl.pallas_call(
        _proj_body,
        grid=(N // BN,),
        in_specs=[
            pl.BlockSpec((BN, D), lambda i: (i, 0)),
            pl.BlockSpec((3 * D, D), lambda i: (0, 0)),
            pl.BlockSpec((1, D), lambda i: (0, 0)),
        ],
        out_specs=[pl.BlockSpec((BN, D), lambda i: (i, 0))] * 2,
        out_shape=[jax.ShapeDtypeStruct((N, D), jnp.float32)] * 2,
    )(node_features, me_w1, me_b1.reshape(1, D))

    gpairs = [g(xs, xr, snd, rcv) for g in _gather_calls]

    BE = 2000
    EBS = ES // BE
    w1e = me_w1[2 * D:3 * D]
    b2 = me_b2.reshape(1, D)

    def _edge_specs(s):
        return [
            pl.BlockSpec((BE, D), lambda i: (i, 0)),
            pl.BlockSpec((BE, D), lambda i: (i, 0)),
            pl.BlockSpec((BE, D), lambda i, s=s: (s * EBS + i, 0)),
            pl.BlockSpec((D, D), lambda i: (0, 0)),
            pl.BlockSpec((D, D), lambda i: (0, 0)),
            pl.BlockSpec((1, D), lambda i: (0, 0)),
        ]

    ne0, eo = pl.pallas_call(
        _edge_body,
        grid=(EBS,),
        in_specs=_edge_specs(0),
        out_specs=[
            pl.BlockSpec((BE, D), lambda i: (i, 0)),
            pl.BlockSpec((BE, D), lambda i: (i, 0)),
        ],
        out_shape=[
            jax.ShapeDtypeStruct((ES, D), jnp.float32),
            jax.ShapeDtypeStruct((E, D), jnp.float32),
        ],
    )(gpairs[0][0], gpairs[0][1], edge_features, w1e, me_w2, b2)

    ne1, eo = pl.pallas_call(
        _edge_body_alias,
        grid=(EBS,),
        in_specs=_edge_specs(1) + [pl.BlockSpec(memory_space=pltpu.HBM)],
        out_specs=[
            pl.BlockSpec((BE, D), lambda i: (i, 0)),
            pl.BlockSpec((BE, D), lambda i: (EBS + i, 0)),
        ],
        out_shape=[
            jax.ShapeDtypeStruct((ES, D), jnp.float32),
            jax.ShapeDtypeStruct((E, D), jnp.float32),
        ],
        input_output_aliases={6: 1},
    )(gpairs[1][0], gpairs[1][1], edge_features, w1e, me_w2, b2, eo)

    zeros = jnp.zeros((NP, D), jnp.float32)
    parts0 = _scatter_calls[0](ne0, rcv, zeros)
    parts1 = _scatter_calls[1](ne1, rcv, zeros)

    node_out = pl.pallas_call(
        _node_body,
        grid=(N // BN,),
        in_specs=[
            pl.BlockSpec((BN, D), lambda i: (i, 0)),
            pl.BlockSpec((NC, BN, D), lambda i: (0, i, 0)),
            pl.BlockSpec((NC, BN, D), lambda i: (0, i, 0)),
            pl.BlockSpec((2 * D, D), lambda i: (0, 0)),
            pl.BlockSpec((1, D), lambda i: (0, 0)),
            pl.BlockSpec((D, D), lambda i: (0, 0)),
            pl.BlockSpec((1, D), lambda i: (0, 0)),
        ],
        out_specs=pl.BlockSpec((BN, D), lambda i: (i, 0)),
        out_shape=jax.ShapeDtypeStruct((N, D), jnp.float32),
    )(node_features, parts0, parts1, nm_w1, nm_b1.reshape(1, D), nm_w2,
      nm_b2.reshape(1, D))

    return node_out, eo


# speedup vs baseline: 1.0037x; 1.0037x over previous
import functools

import jax
import jax.numpy as jnp
from jax import lax
from jax.experimental import pallas as pl
from jax.experimental.pallas import tpu as pltpu
from jax.experimental.pallas import tpu_sc as plsc

N = 10000
E = 320000
D = 128

NC = 2
NS = 16
NW = NC * NS
EPW = E // NW
CHUNK = 80
NCHUNK = EPW // CHUNK
NBUF = 5
NOUTER = NCHUNK // NBUF
NP = 10240
RPS = NP // NS
SCHUNK = 40
SNCHUNK = EPW // SCHUNK
SNOUTER = SNCHUNK // NBUF
RSTEP = SCHUNK

_mesh = plsc.VectorSubcoreMesh(core_axis_name="c", subcore_axis_name="s")


def _proj_body(x_ref, w1_ref, b1_ref, xs_ref, xr_ref):
    x = x_ref[...]
    xs_ref[...] = (
        jnp.dot(x, w1_ref[0:D, :], preferred_element_type=jnp.float32)
        + b1_ref[...]
    )
    xr_ref[...] = jnp.dot(x, w1_ref[D:2 * D, :], preferred_element_type=jnp.float32)


def _edge_body(s_ref, e_ref, w1e_ref, w2_ref, b2_ref, ne_ref, eo_ref):
    e = e_ref[...]
    pe = jnp.dot(e, w1e_ref[...], preferred_element_type=jnp.float32)
    h = jnp.maximum(s_ref[...] + pe, 0.0)
    tmp = jnp.dot(h, w2_ref[...], preferred_element_type=jnp.float32) + b2_ref[...]
    ne_ref[...] = tmp
    eo_ref[...] = tmp + e


def _node_body(x_ref, p_ref, w1_ref, b1_ref, w2_ref, b2_ref, out_ref):
    x = x_ref[...]
    agg = p_ref[0] + p_ref[1]
    h = jnp.maximum(
        jnp.dot(x, w1_ref[0:D, :], preferred_element_type=jnp.float32)
        + jnp.dot(agg, w1_ref[D:2 * D, :], preferred_element_type=jnp.float32)
        + b1_ref[...],
        0.0,
    )
    out_ref[...] = (
        jnp.dot(h, w2_ref[...], preferred_element_type=jnp.float32)
        + b2_ref[...]
        + x
    )


@functools.partial(
    pl.kernel,
    mesh=_mesh,
    out_type=jax.ShapeDtypeStruct((E, D), jnp.float32),
    scratch_types=(
        [pltpu.VMEM((CHUNK,), jnp.int32) for _ in range(2 * NBUF)]
        + [pltpu.VMEM((CHUNK, D), jnp.float32) for _ in range(2 * NBUF)]
        + [pltpu.SemaphoreType.DMA for _ in range(3 * NBUF)]
    ),
)
def _gather_sc(xs_hbm, xr_hbm, snd_hbm, rcv_hbm, s_hbm, *scr):
    idx_s = scr[0:NBUF]
    idx_r = scr[NBUF:2 * NBUF]
    rows_s = scr[2 * NBUF:3 * NBUF]
    rows_r = scr[3 * NBUF:4 * NBUF]
    sem_i = scr[4 * NBUF:5 * NBUF]
    sem_g = scr[5 * NBUF:6 * NBUF]
    sem_w = scr[6 * NBUF:7 * NBUF]

    wid = lax.axis_index("s") * NC + lax.axis_index("c")
    base = wid * EPW

    for b in range(NBUF):
        off = base + b * CHUNK
        pltpu.async_copy(snd_hbm.at[pl.ds(off, CHUNK)], idx_s[b], sem_i[b])
        pltpu.async_copy(rcv_hbm.at[pl.ds(off, CHUNK)], idx_r[b], sem_i[b])

    def outer(g, carry):
        for b in range(NBUF):
            k = g * NBUF + b
            off = base + k * CHUNK

            @pl.when(g > 0)
            def _drain():
                pltpu.make_async_copy(
                    rows_s[b], s_hbm.at[pl.ds(base, CHUNK)], sem_w[b]).wait()

            pltpu.make_async_copy(
                snd_hbm.at[pl.ds(base, CHUNK)], idx_s[b], sem_i[b]).wait()
            pltpu.make_async_copy(
                rcv_hbm.at[pl.ds(base, CHUNK)], idx_r[b], sem_i[b]).wait()
            cs = pltpu.async_copy(xs_hbm.at[idx_s[b]], rows_s[b], sem_g[b])
            cr = pltpu.async_copy(xr_hbm.at[idx_r[b]], rows_r[b], sem_g[b])
            cs.wait()
            cr.wait()

            def vadd(r, carry):
                for j in range(D // 16):
                    sl = pl.ds(j * 16, 16)
                    rows_s[b][r, sl] = rows_s[b][r, sl] + rows_r[b][r, sl]
                return carry

            lax.fori_loop(0, CHUNK, vadd, 0)
            pltpu.async_copy(rows_s[b], s_hbm.at[pl.ds(off, CHUNK)], sem_w[b])

            @pl.when(k + NBUF < NCHUNK)
            def _prefetch():
                off2 = off + NBUF * CHUNK
                pltpu.async_copy(snd_hbm.at[pl.ds(off2, CHUNK)], idx_s[b], sem_i[b])
                pltpu.async_copy(rcv_hbm.at[pl.ds(off2, CHUNK)], idx_r[b], sem_i[b])
        return carry

    lax.fori_loop(0, NOUTER, outer, 0)
    for b in range(NBUF):
        pltpu.make_async_copy(
            rows_s[b], s_hbm.at[pl.ds(base, CHUNK)], sem_w[b]).wait()


@functools.partial(
    pl.kernel,
    mesh=_mesh,
    out_type=jax.ShapeDtypeStruct((NC, NP, D), jnp.float32),
    scratch_types=(
        [pltpu.VMEM((SCHUNK,), jnp.int32) for _ in range(NBUF)]
        + [pltpu.VMEM((SCHUNK, D), jnp.float32) for _ in range(NBUF)]
        + [pltpu.VMEM_SHARED((NP, D), jnp.float32)]
        + [pltpu.SemaphoreType.DMA for _ in range(NBUF)]
    ),
)
def _scatter_sc(ne_hbm, rcv_hbm, zero_hbm, parts_hbm, *scr):
    idx = scr[0:NBUF]
    chunk = scr[NBUF:2 * NBUF]
    acc = scr[2 * NBUF]
    sem_i = scr[2 * NBUF + 1:3 * NBUF + 1]

    cid = lax.axis_index("c")
    sid = lax.axis_index("s")
    wid = sid * NC + cid

    pltpu.sync_copy(zero_hbm.at[pl.ds(0, RSTEP)], chunk[0])

    def zbody(i, carry):
        r0 = sid * RPS + i * RSTEP
        pltpu.sync_copy(chunk[0], acc.at[pl.ds(r0, RSTEP)])
        return carry

    lax.fori_loop(0, RPS // RSTEP, zbody, 0)
    plsc.subcore_barrier()

    base = wid * EPW

    for b in range(NBUF):
        off = base + b * SCHUNK
        pltpu.async_copy(rcv_hbm.at[pl.ds(off, SCHUNK)], idx[b], sem_i[b])
        pltpu.async_copy(ne_hbm.at[pl.ds(off, SCHUNK)], chunk[b], sem_i[b])

    def outer(g, carry):
        for b in range(NBUF):
            k = g * NBUF + b
            off = base + k * SCHUNK
            pltpu.make_async_copy(
                rcv_hbm.at[pl.ds(base, SCHUNK)], idx[b], sem_i[b]).wait()
            pltpu.make_async_copy(
                ne_hbm.at[pl.ds(base, SCHUNK)], chunk[b], sem_i[b]).wait()
            pltpu.sync_copy(chunk[b], acc.at[idx[b]], add=True)

            @pl.when(k + NBUF < SNCHUNK)
            def _prefetch():
                off2 = off + NBUF * SCHUNK
                pltpu.async_copy(rcv_hbm.at[pl.ds(off2, SCHUNK)], idx[b], sem_i[b])
                pltpu.async_copy(ne_hbm.at[pl.ds(off2, SCHUNK)], chunk[b], sem_i[b])
        return carry

    lax.fori_loop(0, SNOUTER, outer, 0)
    plsc.subcore_barrier()

    def obody(i, carry):
        r0 = sid * RPS + i * RSTEP
        pltpu.sync_copy(acc.at[pl.ds(r0, RSTEP)], chunk[0])
        pltpu.sync_copy(chunk[0], parts_hbm.at[cid, pl.ds(r0, RSTEP)])
        return carry

    lax.fori_loop(0, RPS // RSTEP, obody, 0)


def kernel(node_features, edge_features, me_w1, me_b1, me_w2, me_b2,
           nm_w1, nm_b1, nm_w2, nm_b2, senders, receivers):
    snd = senders.astype(jnp.int32)
    rcv = receivers.astype(jnp.int32)

    BN = 1000
    xs, xr = pl.pallas_call(
        _proj_body,
        grid=(N // BN,),
        in_specs=[
            pl.BlockSpec((BN, D), lambda i: (i, 0)),
            pl.BlockSpec((3 * D, D), lambda i: (0, 0)),
            pl.BlockSpec((1, D), lambda i: (0, 0)),
        ],
        out_specs=[pl.BlockSpec((BN, D), lambda i: (i, 0))] * 2,
        out_shape=[jax.ShapeDtypeStruct((N, D), jnp.float32)] * 2,
    )(node_features, me_w1, me_b1.reshape(1, D))

    s = _gather_sc(xs, xr, snd, rcv)

    BE = 2000
    ne, edge_out = pl.pallas_call(
        _edge_body,
        grid=(E // BE,),
        in_specs=[
            pl.BlockSpec((BE, D), lambda i: (i, 0)),
            pl.BlockSpec((BE, D), lambda i: (i, 0)),
            pl.BlockSpec((D, D), lambda i: (0, 0)),
            pl.BlockSpec((D, D), lambda i: (0, 0)),
            pl.BlockSpec((1, D), lambda i: (0, 0)),
        ],
        out_specs=[pl.BlockSpec((BE, D), lambda i: (i, 0))] * 2,
        out_shape=[jax.ShapeDtypeStruct((E, D), jnp.float32)] * 2,
    )(s, edge_features, me_w1[2 * D:3 * D], me_w2, me_b2.reshape(1, D))

    zeros = jnp.zeros((NP, D), jnp.float32)
    parts = _scatter_sc(ne, rcv, zeros)

    node_out = pl.pallas_call(
        _node_body,
        grid=(N // BN,),
        in_specs=[
            pl.BlockSpec((BN, D), lambda i: (i, 0)),
            pl.BlockSpec((NC, BN, D), lambda i: (0, i, 0)),
            pl.BlockSpec((2 * D, D), lambda i: (0, 0)),
            pl.BlockSpec((1, D), lambda i: (0, 0)),
            pl.BlockSpec((D, D), lambda i: (0, 0)),
            pl.BlockSpec((1, D), lambda i: (0, 0)),
        ],
        out_specs=pl.BlockSpec((BN, D), lambda i: (i, 0)),
        out_shape=jax.ShapeDtypeStruct((N, D), jnp.float32),
    )(node_features, parts, nm_w1, nm_b1.reshape(1, D), nm_w2, nm_b2.reshape(1, D))

    return node_out, edge_out


# speedup vs baseline: 1.1064x; 1.1023x over previous
import functools

import jax
import jax.numpy as jnp
from jax import lax
from jax.experimental import pallas as pl
from jax.experimental.pallas import tpu as pltpu
from jax.experimental.pallas import tpu_sc as plsc

N = 10000
E = 320000
D = 128

NC = 2
NS = 16
NW = NC * NS
EPW = E // NW
NBUF = 5
GCHUNK = 40
GNCHUNK = EPW // GCHUNK
NRING = 10
GOFF = 5
NGOUT = GNCHUNK // NRING
NP = 10240
RPS = NP // NS
SCHUNK = 40
SNCHUNK = EPW // SCHUNK
SNOUTER = SNCHUNK // NBUF
RSTEP = SCHUNK

_mesh = plsc.VectorSubcoreMesh(core_axis_name="c", subcore_axis_name="s")


def _proj_body(x_ref, w1_ref, b1_ref, xs_ref, xr_ref):
    x = x_ref[...]
    xs_ref[...] = (
        jnp.dot(x, w1_ref[0:D, :], preferred_element_type=jnp.float32)
        + b1_ref[...]
    )
    xr_ref[...] = jnp.dot(x, w1_ref[D:2 * D, :], preferred_element_type=jnp.float32)


def _edge_body(gs_ref, gr_ref, e_ref, w1e_ref, w2_ref, b2_ref, ne_ref, eo_ref):
    e = e_ref[...]
    pe = jnp.dot(e, w1e_ref[...], preferred_element_type=jnp.float32)
    h = jnp.maximum(gs_ref[...] + gr_ref[...] + pe, 0.0)
    tmp = jnp.dot(h, w2_ref[...], preferred_element_type=jnp.float32) + b2_ref[...]
    ne_ref[...] = tmp
    eo_ref[...] = tmp + e


def _node_body(x_ref, p_ref, w1_ref, b1_ref, w2_ref, b2_ref, out_ref):
    x = x_ref[...]
    agg = p_ref[0] + p_ref[1]
    h = jnp.maximum(
        jnp.dot(x, w1_ref[0:D, :], preferred_element_type=jnp.float32)
        + jnp.dot(agg, w1_ref[D:2 * D, :], preferred_element_type=jnp.float32)
        + b1_ref[...],
        0.0,
    )
    out_ref[...] = (
        jnp.dot(h, w2_ref[...], preferred_element_type=jnp.float32)
        + b2_ref[...]
        + x
    )


@functools.partial(
    pl.kernel,
    mesh=_mesh,
    out_type=[
        jax.ShapeDtypeStruct((E, D), jnp.float32),
        jax.ShapeDtypeStruct((E, D), jnp.float32),
    ],
    scratch_types=(
        [pltpu.VMEM((GCHUNK,), jnp.int32) for _ in range(2 * NRING)]
        + [pltpu.VMEM((GCHUNK, D), jnp.float32) for _ in range(2 * NRING)]
        + [pltpu.SemaphoreType.DMA for _ in range(3 * NRING)]
    ),
)
def _gather_sc(xs_hbm, xr_hbm, snd_hbm, rcv_hbm, gs_hbm, gr_hbm, *scr):
    idx_s = scr[0:NRING]
    idx_r = scr[NRING:2 * NRING]
    rows_s = scr[2 * NRING:3 * NRING]
    rows_r = scr[3 * NRING:4 * NRING]
    sem_i = scr[4 * NRING:5 * NRING]
    sem_g = scr[5 * NRING:6 * NRING]
    sem_w = scr[6 * NRING:7 * NRING]

    wid = lax.axis_index("s") * NC + lax.axis_index("c")
    base = wid * EPW

    def _front(k, b):
        pltpu.make_async_copy(
            snd_hbm.at[pl.ds(base, GCHUNK)], idx_s[b], sem_i[b]).wait()
        pltpu.make_async_copy(
            rcv_hbm.at[pl.ds(base, GCHUNK)], idx_r[b], sem_i[b]).wait()
        pltpu.async_copy(xs_hbm.at[idx_s[b]], rows_s[b], sem_g[b])
        pltpu.async_copy(xr_hbm.at[idx_r[b]], rows_r[b], sem_g[b])

    def _back(j, b):
        off = base + j * GCHUNK
        pltpu.make_async_copy(
            xs_hbm.at[idx_s[b]], rows_s[b], sem_g[b]).wait()
        pltpu.make_async_copy(
            xr_hbm.at[idx_r[b]], rows_r[b], sem_g[b]).wait()
        pltpu.async_copy(rows_s[b], gs_hbm.at[pl.ds(off, GCHUNK)], sem_w[b])
        pltpu.async_copy(rows_r[b], gr_hbm.at[pl.ds(off, GCHUNK)], sem_w[b])

        @pl.when(j + NRING < GNCHUNK)
        def _prefetch():
            ioff = base + (j + NRING) * GCHUNK
            pltpu.async_copy(snd_hbm.at[pl.ds(ioff, GCHUNK)], idx_s[b], sem_i[b])
            pltpu.async_copy(rcv_hbm.at[pl.ds(ioff, GCHUNK)], idx_r[b], sem_i[b])

    for b in range(NRING):
        off = base + b * GCHUNK
        pltpu.async_copy(snd_hbm.at[pl.ds(off, GCHUNK)], idx_s[b], sem_i[b])
        pltpu.async_copy(rcv_hbm.at[pl.ds(off, GCHUNK)], idx_r[b], sem_i[b])

    def outer(g, carry):
        for b in range(NRING):
            k = g * NRING + b
            j = k - GOFF
            bj = (b + NRING - GOFF) % NRING

            @pl.when(g > 0)
            def _drain():
                pltpu.make_async_copy(
                    rows_s[b], gs_hbm.at[pl.ds(base, GCHUNK)], sem_w[b]).wait()
                pltpu.make_async_copy(
                    rows_r[b], gr_hbm.at[pl.ds(base, GCHUNK)], sem_w[b]).wait()

            _front(k, b)

            @pl.when(j >= 0)
            def _backstage():
                _back(j, bj)
        return carry

    lax.fori_loop(0, NGOUT, outer, 0)

    for b in range(GOFF):
        _back(GNCHUNK - GOFF + b, (b + NRING - GOFF) % NRING)
    for b in range(NRING):
        pltpu.make_async_copy(
            rows_s[b], gs_hbm.at[pl.ds(base, GCHUNK)], sem_w[b]).wait()
        pltpu.make_async_copy(
            rows_r[b], gr_hbm.at[pl.ds(base, GCHUNK)], sem_w[b]).wait()


@functools.partial(
    pl.kernel,
    mesh=_mesh,
    out_type=jax.ShapeDtypeStruct((NC, NP, D), jnp.float32),
    scratch_types=(
        [pltpu.VMEM((SCHUNK,), jnp.int32) for _ in range(NBUF)]
        + [pltpu.VMEM((SCHUNK, D), jnp.float32) for _ in range(NBUF)]
        + [pltpu.VMEM_SHARED((NP, D), jnp.float32)]
        + [pltpu.SemaphoreType.DMA for _ in range(NBUF)]
    ),
)
def _scatter_sc(ne_hbm, rcv_hbm, zero_hbm, parts_hbm, *scr):
    idx = scr[0:NBUF]
    chunk = scr[NBUF:2 * NBUF]
    acc = scr[2 * NBUF]
    sem_i = scr[2 * NBUF + 1:3 * NBUF + 1]

    cid = lax.axis_index("c")
    sid = lax.axis_index("s")
    wid = sid * NC + cid

    pltpu.sync_copy(zero_hbm.at[pl.ds(0, RSTEP)], chunk[0])

    def zbody(i, carry):
        r0 = sid * RPS + i * RSTEP
        pltpu.sync_copy(chunk[0], acc.at[pl.ds(r0, RSTEP)])
        return carry

    lax.fori_loop(0, RPS // RSTEP, zbody, 0)
    plsc.subcore_barrier()

    base = wid * EPW

    for b in range(NBUF):
        off = base + b * SCHUNK
        pltpu.async_copy(rcv_hbm.at[pl.ds(off, SCHUNK)], idx[b], sem_i[b])
        pltpu.async_copy(ne_hbm.at[pl.ds(off, SCHUNK)], chunk[b], sem_i[b])

    def outer(g, carry):
        for b in range(NBUF):
            k = g * NBUF + b
            off = base + k * SCHUNK
            pltpu.make_async_copy(
                rcv_hbm.at[pl.ds(base, SCHUNK)], idx[b], sem_i[b]).wait()
            pltpu.make_async_copy(
                ne_hbm.at[pl.ds(base, SCHUNK)], chunk[b], sem_i[b]).wait()
            pltpu.sync_copy(chunk[b], acc.at[idx[b]], add=True)

            @pl.when(k + NBUF < SNCHUNK)
            def _prefetch():
                off2 = off + NBUF * SCHUNK
                pltpu.async_copy(rcv_hbm.at[pl.ds(off2, SCHUNK)], idx[b], sem_i[b])
                pltpu.async_copy(ne_hbm.at[pl.ds(off2, SCHUNK)], chunk[b], sem_i[b])
        return carry

    lax.fori_loop(0, SNOUTER, outer, 0)
    plsc.subcore_barrier()

    def obody(i, carry):
        r0 = sid * RPS + i * RSTEP
        pltpu.sync_copy(acc.at[pl.ds(r0, RSTEP)], chunk[0])
        pltpu.sync_copy(chunk[0], parts_hbm.at[cid, pl.ds(r0, RSTEP)])
        return carry

    lax.fori_loop(0, RPS // RSTEP, obody, 0)


def kernel(node_features, edge_features, me_w1, me_b1, me_w2, me_b2,
           nm_w1, nm_b1, nm_w2, nm_b2, senders, receivers):
    snd = senders.astype(jnp.int32)
    rcv = receivers.astype(jnp.int32)

    BN = 1000
    xs, xr = pl.pallas_call(
        _proj_body,
        grid=(N // BN,),
        in_specs=[
            pl.BlockSpec((BN, D), lambda i: (i, 0)),
            pl.BlockSpec((3 * D, D), lambda i: (0, 0)),
            pl.BlockSpec((1, D), lambda i: (0, 0)),
        ],
        out_specs=[pl.BlockSpec((BN, D), lambda i: (i, 0))] * 2,
        out_shape=[jax.ShapeDtypeStruct((N, D), jnp.float32)] * 2,
    )(node_features, me_w1, me_b1.reshape(1, D))

    gs, gr = _gather_sc(xs, xr, snd, rcv)

    BE = 2000
    ne, edge_out = pl.pallas_call(
        _edge_body,
        grid=(E // BE,),
        in_specs=[
            pl.BlockSpec((BE, D), lambda i: (i, 0)),
            pl.BlockSpec((BE, D), lambda i: (i, 0)),
            pl.BlockSpec((BE, D), lambda i: (i, 0)),
            pl.BlockSpec((D, D), lambda i: (0, 0)),
            pl.BlockSpec((D, D), lambda i: (0, 0)),
            pl.BlockSpec((1, D), lambda i: (0, 0)),
        ],
        out_specs=[pl.BlockSpec((BE, D), lambda i: (i, 0))] * 2,
        out_shape=[jax.ShapeDtypeStruct((E, D), jnp.float32)] * 2,
    )(gs, gr, edge_features, me_w1[2 * D:3 * D], me_w2, me_b2.reshape(1, D))

    zeros = jnp.zeros((NP, D), jnp.float32)
    parts = _scatter_sc(ne, rcv, zeros)

    node_out = pl.pallas_call(
        _node_body,
        grid=(N // BN,),
        in_specs=[
            pl.BlockSpec((BN, D), lambda i: (i, 0)),
            pl.BlockSpec((NC, BN, D), lambda i: (0, i, 0)),
            pl.BlockSpec((2 * D, D), lambda i: (0, 0)),
            pl.BlockSpec((1, D), lambda i: (0, 0)),
            pl.BlockSpec((D, D), lambda i: (0, 0)),
            pl.BlockSpec((1, D), lambda i: (0, 0)),
        ],
        out_specs=pl.BlockSpec((BN, D), lambda i: (i, 0)),
        out_shape=jax.ShapeDtypeStruct((N, D), jnp.float32),
    )(node_features, parts, nm_w1, nm_b1.reshape(1, D), nm_w2, nm_b2.reshape(1, D))

    return node_out, edge_out


# speedup vs baseline: 1.1292x; 1.0207x over previous
import functools

import jax
import jax.numpy as jnp
from jax import lax
from jax.experimental import pallas as pl
from jax.experimental.pallas import tpu as pltpu
from jax.experimental.pallas import tpu_sc as plsc

N = 10000
E = 320000
D = 128

NSPLIT = 2
ES = E // NSPLIT
NC = 2
NS = 16
NW = NC * NS
EPW = ES // NW

GCHUNK = 40
GNCHUNK = EPW // GCHUNK
NRING = 10
GOFF = 5
NGOUT = GNCHUNK // NRING
GTAIL = GNCHUNK - NGOUT * NRING

NBUF = 5
SCHUNK = 40
SNCHUNK = EPW // SCHUNK
SNOUTER = SNCHUNK // NBUF
NP = 10240
RPS = NP // NS
RSTEP = SCHUNK

_mesh = plsc.VectorSubcoreMesh(core_axis_name="c", subcore_axis_name="s")


def _proj_body(x_ref, w1_ref, b1_ref, xs_ref, xr_ref):
    x = x_ref[...]
    xs_ref[...] = (
        jnp.dot(x, w1_ref[0:D, :], preferred_element_type=jnp.float32)
        + b1_ref[...]
    )
    xr_ref[...] = jnp.dot(x, w1_ref[D:2 * D, :], preferred_element_type=jnp.float32)


def _edge_body(gs_ref, gr_ref, e_ref, w1e_ref, w2_ref, b2_ref, ne_ref, eo_ref):
    e = e_ref[...]
    pe = jnp.dot(e, w1e_ref[...], preferred_element_type=jnp.float32)
    h = jnp.maximum(gs_ref[...] + gr_ref[...] + pe, 0.0)
    tmp = jnp.dot(h, w2_ref[...], preferred_element_type=jnp.float32) + b2_ref[...]
    ne_ref[...] = tmp
    eo_ref[...] = tmp + e


def _edge_body_alias(gs_ref, gr_ref, e_ref, w1e_ref, w2_ref, b2_ref, eo_in_ref,
                     ne_ref, eo_ref):
    del eo_in_ref
    _edge_body(gs_ref, gr_ref, e_ref, w1e_ref, w2_ref, b2_ref, ne_ref, eo_ref)


def _node_body(x_ref, p0_ref, p1_ref, w1_ref, b1_ref, w2_ref, b2_ref, out_ref):
    x = x_ref[...]
    agg = (p0_ref[0] + p0_ref[1]) + (p1_ref[0] + p1_ref[1])
    h = jnp.maximum(
        jnp.dot(x, w1_ref[0:D, :], preferred_element_type=jnp.float32)
        + jnp.dot(agg, w1_ref[D:2 * D, :], preferred_element_type=jnp.float32)
        + b1_ref[...],
        0.0,
    )
    out_ref[...] = (
        jnp.dot(h, w2_ref[...], preferred_element_type=jnp.float32)
        + b2_ref[...]
        + x
    )


def _make_gather(soff):
    @functools.partial(
        pl.kernel,
        mesh=_mesh,
        out_type=[
            jax.ShapeDtypeStruct((ES, D), jnp.float32),
            jax.ShapeDtypeStruct((ES, D), jnp.float32),
        ],
        scratch_types=(
            [pltpu.VMEM((GCHUNK,), jnp.int32) for _ in range(2 * NRING)]
            + [pltpu.VMEM((GCHUNK, D), jnp.float32) for _ in range(2 * NRING)]
            + [pltpu.SemaphoreType.DMA for _ in range(3 * NRING)]
        ),
    )
    def gather_sc(xs_hbm, xr_hbm, snd_hbm, rcv_hbm, gs_hbm, gr_hbm, *scr):
        idx_s = scr[0:NRING]
        idx_r = scr[NRING:2 * NRING]
        rows_s = scr[2 * NRING:3 * NRING]
        rows_r = scr[3 * NRING:4 * NRING]
        sem_i = scr[4 * NRING:5 * NRING]
        sem_g = scr[5 * NRING:6 * NRING]
        sem_w = scr[6 * NRING:7 * NRING]

        wid = lax.axis_index("s") * NC + lax.axis_index("c")
        base = wid * EPW
        ibase = soff + base

        def _drain_wb(b):
            pltpu.make_async_copy(
                rows_s[b], gs_hbm.at[pl.ds(base, GCHUNK)], sem_w[b]).wait()
            pltpu.make_async_copy(
                rows_r[b], gr_hbm.at[pl.ds(base, GCHUNK)], sem_w[b]).wait()

        def _front(b):
            pltpu.make_async_copy(
                snd_hbm.at[pl.ds(ibase, GCHUNK)], idx_s[b], sem_i[b]).wait()
            pltpu.make_async_copy(
                rcv_hbm.at[pl.ds(ibase, GCHUNK)], idx_r[b], sem_i[b]).wait()
            pltpu.async_copy(xs_hbm.at[idx_s[b]], rows_s[b], sem_g[b])
            pltpu.async_copy(xr_hbm.at[idx_r[b]], rows_r[b], sem_g[b])

        def _back(j, b, prefetch):
            off = base + j * GCHUNK
            pltpu.make_async_copy(
                xs_hbm.at[idx_s[b]], rows_s[b], sem_g[b]).wait()
            pltpu.make_async_copy(
                xr_hbm.at[idx_r[b]], rows_r[b], sem_g[b]).wait()
            pltpu.async_copy(rows_s[b], gs_hbm.at[pl.ds(off, GCHUNK)], sem_w[b])
            pltpu.async_copy(rows_r[b], gr_hbm.at[pl.ds(off, GCHUNK)], sem_w[b])
            if prefetch:
                @pl.when(j + NRING < GNCHUNK)
                def _prefetch():
                    ioff = ibase + (j + NRING) * GCHUNK
                    pltpu.async_copy(
                        snd_hbm.at[pl.ds(ioff, GCHUNK)], idx_s[b], sem_i[b])
                    pltpu.async_copy(
                        rcv_hbm.at[pl.ds(ioff, GCHUNK)], idx_r[b], sem_i[b])

        for b in range(NRING):
            ioff = ibase + b * GCHUNK
            pltpu.async_copy(snd_hbm.at[pl.ds(ioff, GCHUNK)], idx_s[b], sem_i[b])
            pltpu.async_copy(rcv_hbm.at[pl.ds(ioff, GCHUNK)], idx_r[b], sem_i[b])

        def outer(g, carry):
            for b in range(NRING):
                k = g * NRING + b
                j = k - GOFF
                bj = (b + NRING - GOFF) % NRING

                @pl.when(g > 0)
                def _drain():
                    _drain_wb(b)

                _front(b)

                @pl.when(j >= 0)
                def _backstage():
                    _back(j, bj, prefetch=True)
            return carry

        lax.fori_loop(0, NGOUT, outer, 0)

        k0 = NGOUT * NRING
        for b in range(GTAIL):
            _drain_wb(b)
            _front(b)
            _back(k0 - GOFF + b, (b + NRING - GOFF) % NRING, prefetch=False)
        for b in range(GOFF):
            jj = GNCHUNK - GOFF + b
            _back(jj, jj % NRING, prefetch=False)
        for b in range(NRING):
            _drain_wb(b)

    return gather_sc


def _make_scatter(soff):
    @functools.partial(
        pl.kernel,
        mesh=_mesh,
        out_type=jax.ShapeDtypeStruct((NC, NP, D), jnp.float32),
        scratch_types=(
            [pltpu.VMEM((SCHUNK,), jnp.int32) for _ in range(NBUF)]
            + [pltpu.VMEM((SCHUNK, D), jnp.float32) for _ in range(NBUF)]
            + [pltpu.VMEM_SHARED((NP, D), jnp.float32)]
            + [pltpu.SemaphoreType.DMA for _ in range(NBUF)]
        ),
    )
    def scatter_sc(ne_hbm, rcv_hbm, zero_hbm, parts_hbm, *scr):
        idx = scr[0:NBUF]
        chunk = scr[NBUF:2 * NBUF]
        acc = scr[2 * NBUF]
        sem_i = scr[2 * NBUF + 1:3 * NBUF + 1]

        cid = lax.axis_index("c")
        sid = lax.axis_index("s")
        wid = sid * NC + cid

        pltpu.sync_copy(zero_hbm.at[pl.ds(0, RSTEP)], chunk[0])

        def zbody(i, carry):
            r0 = sid * RPS + i * RSTEP
            pltpu.sync_copy(chunk[0], acc.at[pl.ds(r0, RSTEP)])
            return carry

        lax.fori_loop(0, RPS // RSTEP, zbody, 0)
        plsc.subcore_barrier()

        base = wid * EPW
        ibase = soff + base

        for b in range(NBUF):
            pltpu.async_copy(
                rcv_hbm.at[pl.ds(ibase + b * SCHUNK, SCHUNK)], idx[b], sem_i[b])
            pltpu.async_copy(
                ne_hbm.at[pl.ds(base + b * SCHUNK, SCHUNK)], chunk[b], sem_i[b])

        def outer(g, carry):
            for b in range(NBUF):
                k = g * NBUF + b
                pltpu.make_async_copy(
                    rcv_hbm.at[pl.ds(ibase, SCHUNK)], idx[b], sem_i[b]).wait()
                pltpu.make_async_copy(
                    ne_hbm.at[pl.ds(base, SCHUNK)], chunk[b], sem_i[b]).wait()
                pltpu.sync_copy(chunk[b], acc.at[idx[b]], add=True)

                @pl.when(k + NBUF < SNCHUNK)
                def _prefetch():
                    k2 = k + NBUF
                    pltpu.async_copy(
                        rcv_hbm.at[pl.ds(ibase + k2 * SCHUNK, SCHUNK)],
                        idx[b], sem_i[b])
                    pltpu.async_copy(
                        ne_hbm.at[pl.ds(base + k2 * SCHUNK, SCHUNK)],
                        chunk[b], sem_i[b])
            return carry

        lax.fori_loop(0, SNOUTER, outer, 0)
        plsc.subcore_barrier()

        def obody(i, carry):
            r0 = sid * RPS + i * RSTEP
            pltpu.sync_copy(acc.at[pl.ds(r0, RSTEP)], chunk[0])
            pltpu.sync_copy(chunk[0], parts_hbm.at[cid, pl.ds(r0, RSTEP)])
            return carry

        lax.fori_loop(0, RPS // RSTEP, obody, 0)

    return scatter_sc


_gather_calls = [_make_gather(s * ES) for s in range(NSPLIT)]
_scatter_calls = [_make_scatter(s * ES) for s in range(NSPLIT)]


def kernel(node_features, edge_features, me_w1, me_b1, me_w2, me_b2,
           nm_w1, nm_b1, nm_w2, nm_b2, senders, receivers):
    snd = senders.astype(jnp.int32)
    rcv = receivers.astype(jnp.int32)

    BN = 1000
    xs, xr = pl.pallas_call(
        _proj_body,
        grid=(N // BN,),
        in_specs=[
            pl.BlockSpec((BN, D), lambda i: (i, 0)),
            pl.BlockSpec((3 * D, D), lambda i: (0, 0)),
            pl.BlockSpec((1, D), lambda i: (0, 0)),
        ],
        out_specs=[pl.BlockSpec((BN, D), lambda i: (i, 0))] * 2,
        out_shape=[jax.ShapeDtypeStruct((N, D), jnp.float32)] * 2,
    )(node_features, me_w1, me_b1.reshape(1, D))

    gpairs = [g(xs, xr, snd, rcv) for g in _gather_calls]

    BE = 2000
    EBS = ES // BE
    w1e = me_w1[2 * D:3 * D]
    b2 = me_b2.reshape(1, D)

    def _edge_specs(s):
        return [
            pl.BlockSpec((BE, D), lambda i: (i, 0)),
            pl.BlockSpec((BE, D), lambda i: (i, 0)),
            pl.BlockSpec((BE, D), lambda i, s=s: (s * EBS + i, 0)),
            pl.BlockSpec((D, D), lambda i: (0, 0)),
            pl.BlockSpec((D, D), lambda i: (0, 0)),
            pl.BlockSpec((1, D), lambda i: (0, 0)),
        ]

    ne0, eo = pl.pallas_call(
        _edge_body,
        grid=(EBS,),
        in_specs=_edge_specs(0),
        out_specs=[
            pl.BlockSpec((BE, D), lambda i: (i, 0)),
            pl.BlockSpec((BE, D), lambda i: (i, 0)),
        ],
        out_shape=[
            jax.ShapeDtypeStruct((ES, D), jnp.float32),
            jax.ShapeDtypeStruct((E, D), jnp.float32),
        ],
    )(gpairs[0][0], gpairs[0][1], edge_features, w1e, me_w2, b2)

    ne1, eo = pl.pallas_call(
        _edge_body_alias,
        grid=(EBS,),
        in_specs=_edge_specs(1) + [pl.BlockSpec(memory_space=pltpu.HBM)],
        out_specs=[
            pl.BlockSpec((BE, D), lambda i: (i, 0)),
            pl.BlockSpec((BE, D), lambda i: (EBS + i, 0)),
        ],
        out_shape=[
            jax.ShapeDtypeStruct((ES, D), jnp.float32),
            jax.ShapeDtypeStruct((E, D), jnp.float32),
        ],
        input_output_aliases={6: 1},
    )(gpairs[1][0], gpairs[1][1], edge_features, w1e, me_w2, b2, eo)

    zeros = jnp.zeros((NP, D), jnp.float32)
    parts0 = _scatter_calls[0](ne0, rcv, zeros)
    parts1 = _scatter_calls[1](ne1, rcv, zeros)

    node_out = pl.pallas_call(
        _node_body,
        grid=(N // BN,),
        in_specs=[
            pl.BlockSpec((BN, D), lambda i: (i, 0)),
            pl.BlockSpec((NC, BN, D), lambda i: (0, i, 0)),
            pl.BlockSpec((NC, BN, D), lambda i: (0, i, 0)),
            pl.BlockSpec((2 * D, D), lambda i: (0, 0)),
            pl.BlockSpec((1, D), lambda i: (0, 0)),
            pl.BlockSpec((D, D), lambda i: (0, 0)),
            pl.BlockSpec((1, D), lambda i: (0, 0)),
        ],
        out_specs=pl.BlockSpec((BN, D), lambda i: (i, 0)),
        out_shape=jax.ShapeDtypeStruct((N, D), jnp.float32),
    )(node_features, parts0, parts1, nm_w1, nm_b1.reshape(1, D), nm_w2,
      nm_b2.reshape(1, D))

    return node_out, eo


# speedup vs baseline: 1.1501x; 1.0185x over previous
import functools

import jax
import jax.numpy as jnp
from jax import lax
from jax.experimental import pallas as pl
from jax.experimental.pallas import tpu as pltpu
from jax.experimental.pallas import tpu_sc as plsc

N = 10000
E = 320000
D = 128

NSPLIT = 2
ES = E // NSPLIT
NC = 2
NS = 16
NW = NC * NS
EPW = ES // NW

GCHUNK = 40
GNCHUNK = EPW // GCHUNK
NRING = 10
GOFF = 5
NGOUT = GNCHUNK // NRING
GTAIL = GNCHUNK - NGOUT * NRING

NBUF = 5
SCHUNK = 40
SNCHUNK = EPW // SCHUNK
SNOUTER = SNCHUNK // NBUF
NP = 10240
RPS = NP // NS
RSTEP = SCHUNK

_mesh = plsc.VectorSubcoreMesh(core_axis_name="c", subcore_axis_name="s")


def _proj_body(x_ref, w1_ref, b1_ref, xs_ref, xr_ref):
    x = x_ref[...]
    xs_ref[...] = (
        jnp.dot(x, w1_ref[0:D, :], preferred_element_type=jnp.float32)
        + b1_ref[...]
    )
    xr_ref[...] = jnp.dot(x, w1_ref[D:2 * D, :], preferred_element_type=jnp.float32)


def _edge_body(gs_ref, gr_ref, e_ref, w1e_ref, w2_ref, b2_ref, ne_ref, eo_ref):
    e = e_ref[...]
    pe = jnp.dot(e, w1e_ref[...], preferred_element_type=jnp.float32)
    h = jnp.maximum(gs_ref[...] + gr_ref[...] + pe, 0.0)
    tmp = jnp.dot(h, w2_ref[...], preferred_element_type=jnp.float32) + b2_ref[...]
    ne_ref[...] = tmp
    eo_ref[...] = tmp + e


def _edge_body_alias(gs_ref, gr_ref, e_ref, w1e_ref, w2_ref, b2_ref, eo_in_ref,
                     ne_ref, eo_ref):
    del eo_in_ref
    _edge_body(gs_ref, gr_ref, e_ref, w1e_ref, w2_ref, b2_ref, ne_ref, eo_ref)


def _node_body(x_ref, p0_ref, p1_ref, w1_ref, b1_ref, w2_ref, b2_ref, out_ref):
    x = x_ref[...]
    agg = (p0_ref[0] + p0_ref[1]) + (p1_ref[0] + p1_ref[1])
    h = jnp.maximum(
        jnp.dot(x, w1_ref[0:D, :], preferred_element_type=jnp.float32)
        + jnp.dot(agg, w1_ref[D:2 * D, :], preferred_element_type=jnp.float32)
        + b1_ref[...],
        0.0,
    )
    out_ref[...] = (
        jnp.dot(h, w2_ref[...], preferred_element_type=jnp.float32)
        + b2_ref[...]
        + x
    )


def _make_gather(soff):
    @functools.partial(
        pl.kernel,
        mesh=_mesh,
        out_type=[
            jax.ShapeDtypeStruct((ES, D), jnp.float32),
            jax.ShapeDtypeStruct((ES, D), jnp.float32),
        ],
        scratch_types=(
            [pltpu.VMEM((GCHUNK,), jnp.int32) for _ in range(2 * NRING)]
            + [pltpu.VMEM((GCHUNK, D), jnp.float32) for _ in range(2 * NRING)]
            + [pltpu.SemaphoreType.DMA for _ in range(3 * NRING)]
        ),
    )
    def gather_sc(xs_hbm, xr_hbm, snd_hbm, rcv_hbm, gs_hbm, gr_hbm, *scr):
        idx_s = scr[0:NRING]
        idx_r = scr[NRING:2 * NRING]
        rows_s = scr[2 * NRING:3 * NRING]
        rows_r = scr[3 * NRING:4 * NRING]
        sem_i = scr[4 * NRING:5 * NRING]
        sem_g = scr[5 * NRING:6 * NRING]
        sem_w = scr[6 * NRING:7 * NRING]

        wid = lax.axis_index("s") * NC + lax.axis_index("c")
        base = wid * EPW
        ibase = soff + base

        def _drain_wb(b):
            pltpu.make_async_copy(
                rows_s[b], gs_hbm.at[pl.ds(base, GCHUNK)], sem_w[b]).wait()
            pltpu.make_async_copy(
                rows_r[b], gr_hbm.at[pl.ds(base, GCHUNK)], sem_w[b]).wait()

        def _front(b):
            pltpu.make_async_copy(
                snd_hbm.at[pl.ds(ibase, GCHUNK)], idx_s[b], sem_i[b]).wait()
            pltpu.make_async_copy(
                rcv_hbm.at[pl.ds(ibase, GCHUNK)], idx_r[b], sem_i[b]).wait()
            pltpu.async_copy(xs_hbm.at[idx_s[b]], rows_s[b], sem_g[b])
            pltpu.async_copy(xr_hbm.at[idx_r[b]], rows_r[b], sem_g[b])

        def _back(j, b, prefetch):
            off = base + j * GCHUNK
            pltpu.make_async_copy(
                xs_hbm.at[idx_s[b]], rows_s[b], sem_g[b]).wait()
            pltpu.make_async_copy(
                xr_hbm.at[idx_r[b]], rows_r[b], sem_g[b]).wait()
            pltpu.async_copy(rows_s[b], gs_hbm.at[pl.ds(off, GCHUNK)], sem_w[b])
            pltpu.async_copy(rows_r[b], gr_hbm.at[pl.ds(off, GCHUNK)], sem_w[b])
            if prefetch:
                @pl.when(j + NRING < GNCHUNK)
                def _prefetch():
                    ioff = ibase + (j + NRING) * GCHUNK
                    pltpu.async_copy(
                        snd_hbm.at[pl.ds(ioff, GCHUNK)], idx_s[b], sem_i[b])
                    pltpu.async_copy(
                        rcv_hbm.at[pl.ds(ioff, GCHUNK)], idx_r[b], sem_i[b])

        for b in range(NRING):
            ioff = ibase + b * GCHUNK
            pltpu.async_copy(snd_hbm.at[pl.ds(ioff, GCHUNK)], idx_s[b], sem_i[b])
            pltpu.async_copy(rcv_hbm.at[pl.ds(ioff, GCHUNK)], idx_r[b], sem_i[b])

        def outer(g, carry):
            for b in range(NRING):
                k = g * NRING + b
                j = k - GOFF
                bj = (b + NRING - GOFF) % NRING

                @pl.when(g > 0)
                def _drain():
                    _drain_wb(b)

                _front(b)

                @pl.when(j >= 0)
                def _backstage():
                    _back(j, bj, prefetch=True)
            return carry

        lax.fori_loop(0, NGOUT, outer, 0)

        k0 = NGOUT * NRING
        for b in range(GTAIL):
            _drain_wb(b)
            _front(b)
            _back(k0 - GOFF + b, (b + NRING - GOFF) % NRING, prefetch=False)
        for b in range(GOFF):
            jj = GNCHUNK - GOFF + b
            _back(jj, jj % NRING, prefetch=False)
        for b in range(NRING):
            _drain_wb(b)

    return gather_sc


def _make_scatter(soff):
    @functools.partial(
        pl.kernel,
        mesh=_mesh,
        out_type=jax.ShapeDtypeStruct((NC, NP, D), jnp.float32),
        scratch_types=(
            [pltpu.VMEM((SCHUNK,), jnp.int32) for _ in range(NBUF)]
            + [pltpu.VMEM((SCHUNK, D), jnp.float32) for _ in range(NBUF)]
            + [pltpu.VMEM_SHARED((NP, D), jnp.float32)]
            + [pltpu.SemaphoreType.DMA for _ in range(NBUF)]
        ),
    )
    def scatter_sc(ne_hbm, rcv_hbm, zero_hbm, parts_hbm, *scr):
        idx = scr[0:NBUF]
        chunk = scr[NBUF:2 * NBUF]
        acc = scr[2 * NBUF]
        sem_i = scr[2 * NBUF + 1:3 * NBUF + 1]

        cid = lax.axis_index("c")
        sid = lax.axis_index("s")
        wid = sid * NC + cid

        pltpu.sync_copy(zero_hbm.at[pl.ds(0, RSTEP)], chunk[0])

        def zbody(i, carry):
            r0 = sid * RPS + i * RSTEP
            pltpu.sync_copy(chunk[0], acc.at[pl.ds(r0, RSTEP)])
            return carry

        lax.fori_loop(0, RPS // RSTEP, zbody, 0)
        plsc.subcore_barrier()

        base = wid * EPW
        ibase = soff + base

        for b in range(NBUF):
            pltpu.async_copy(
                rcv_hbm.at[pl.ds(ibase + b * SCHUNK, SCHUNK)], idx[b], sem_i[b])
            pltpu.async_copy(
                ne_hbm.at[pl.ds(base + b * SCHUNK, SCHUNK)], chunk[b], sem_i[b])

        def outer(g, carry):
            for b in range(NBUF):
                k = g * NBUF + b
                pltpu.make_async_copy(
                    rcv_hbm.at[pl.ds(ibase, SCHUNK)], idx[b], sem_i[b]).wait()
                pltpu.make_async_copy(
                    ne_hbm.at[pl.ds(base, SCHUNK)], chunk[b], sem_i[b]).wait()
                pltpu.sync_copy(chunk[b], acc.at[idx[b]], add=True)

                @pl.when(k + NBUF < SNCHUNK)
                def _prefetch():
                    k2 = k + NBUF
                    pltpu.async_copy(
                        rcv_hbm.at[pl.ds(ibase + k2 * SCHUNK, SCHUNK)],
                        idx[b], sem_i[b])
                    pltpu.async_copy(
                        ne_hbm.at[pl.ds(base + k2 * SCHUNK, SCHUNK)],
                        chunk[b], sem_i[b])
            return carry

        lax.fori_loop(0, SNOUTER, outer, 0)
        plsc.subcore_barrier()

        def obody(i, carry):
            r0 = sid * RPS + i * RSTEP
            pltpu.sync_copy(acc.at[pl.ds(r0, RSTEP)], chunk[0])
            pltpu.sync_copy(chunk[0], parts_hbm.at[cid, pl.ds(r0, RSTEP)])
            return carry

        lax.fori_loop(0, RPS // RSTEP, obody, 0)

    return scatter_sc


_gather_calls = [_make_gather(s * ES) for s in range(NSPLIT)]
_scatter_calls = [_make_scatter(s * ES) for s in range(NSPLIT)]


def kernel(node_features, edge_features, me_w1, me_b1, me_w2, me_b2,
           nm_w1, nm_b1, nm_w2, nm_b2, senders, receivers):
    snd = senders.astype(jnp.int32)
    rcv = receivers.astype(jnp.int32)

    BN = 1000
    xs, xr = pl.pallas_call(
        _proj_body,
        grid=(N // BN,),
        in_specs=[
            pl.BlockSpec((BN, D), lambda i: (i, 0)),
            pl.BlockSpec((3 * D, D), lambda i: (0, 0)),
            pl.BlockSpec((1, D), lambda i: (0, 0)),
        ],
        out_specs=[pl.BlockSpec((BN, D), lambda i: (i, 0))] * 2,
        out_shape=[jax.ShapeDtypeStruct((N, D), jnp.float32)] * 2,
    )(node_features, me_w1, me_b1.reshape(1, D))

    gpairs = [g(xs, xr, snd, rcv) for g in _gather_calls]

    BE = 4000
    EBS = ES // BE
    w1e = me_w1[2 * D:3 * D]
    b2 = me_b2.reshape(1, D)

    def _edge_specs(s):
        return [
            pl.BlockSpec((BE, D), lambda i: (i, 0)),
            pl.BlockSpec((BE, D), lambda i: (i, 0)),
            pl.BlockSpec((BE, D), lambda i, s=s: (s * EBS + i, 0)),
            pl.BlockSpec((D, D), lambda i: (0, 0)),
            pl.BlockSpec((D, D), lambda i: (0, 0)),
            pl.BlockSpec((1, D), lambda i: (0, 0)),
        ]

    ne0, eo = pl.pallas_call(
        _edge_body,
        grid=(EBS,),
        in_specs=_edge_specs(0),
        out_specs=[
            pl.BlockSpec((BE, D), lambda i: (i, 0)),
            pl.BlockSpec((BE, D), lambda i: (i, 0)),
        ],
        out_shape=[
            jax.ShapeDtypeStruct((ES, D), jnp.float32),
            jax.ShapeDtypeStruct((E, D), jnp.float32),
        ],
    )(gpairs[0][0], gpairs[0][1], edge_features, w1e, me_w2, b2)

    ne1, eo = pl.pallas_call(
        _edge_body_alias,
        grid=(EBS,),
        in_specs=_edge_specs(1) + [pl.BlockSpec(memory_space=pltpu.HBM)],
        out_specs=[
            pl.BlockSpec((BE, D), lambda i: (i, 0)),
            pl.BlockSpec((BE, D), lambda i: (EBS + i, 0)),
        ],
        out_shape=[
            jax.ShapeDtypeStruct((ES, D), jnp.float32),
            jax.ShapeDtypeStruct((E, D), jnp.float32),
        ],
        input_output_aliases={6: 1},
    )(gpairs[1][0], gpairs[1][1], edge_features, w1e, me_w2, b2, eo)

    zeros = jnp.zeros((NP, D), jnp.float32)
    parts0 = _scatter_calls[0](ne0, rcv, zeros)
    parts1 = _scatter_calls[1](ne1, rcv, zeros)

    node_out = pl.pallas_call(
        _node_body,
        grid=(N // BN,),
        in_specs=[
            pl.BlockSpec((BN, D), lambda i: (i, 0)),
            pl.BlockSpec((NC, BN, D), lambda i: (0, i, 0)),
            pl.BlockSpec((NC, BN, D), lambda i: (0, i, 0)),
            pl.BlockSpec((2 * D, D), lambda i: (0, 0)),
            pl.BlockSpec((1, D), lambda i: (0, 0)),
            pl.BlockSpec((D, D), lambda i: (0, 0)),
            pl.BlockSpec((1, D), lambda i: (0, 0)),
        ],
        out_specs=pl.BlockSpec((BN, D), lambda i: (i, 0)),
        out_shape=jax.ShapeDtypeStruct((N, D), jnp.float32),
    )(node_features, parts0, parts1, nm_w1, nm_b1.reshape(1, D), nm_w2,
      nm_b2.reshape(1, D))

    return node_out, eo


# speedup vs baseline: 1.3722x; 1.1931x over previous
import functools

import jax
import jax.numpy as jnp
from jax import lax
from jax.experimental import pallas as pl
from jax.experimental.pallas import tpu as pltpu
from jax.experimental.pallas import tpu_sc as plsc

N = 10000
E = 320000
D = 128

NSPLIT = 2
ES = E // NSPLIT
NC = 2
NS = 16
NW = NC * NS
EPW = ES // NW

GCHUNK = 40
GNCHUNK = EPW // GCHUNK
NRING = 10
GOFF = 5
NGOUT = GNCHUNK // NRING
GTAIL = GNCHUNK - NGOUT * NRING

NBUF = 5
SCHUNK = 40
SNCHUNK = EPW // SCHUNK
SNOUTER = SNCHUNK // NBUF
NP = 10240
RPS = NP // NS
RSTEP = SCHUNK

_mesh = plsc.VectorSubcoreMesh(core_axis_name="c", subcore_axis_name="s")


def _proj_body(x_ref, w1_ref, b1_ref, xs_ref, xr_ref):
    x = x_ref[...]
    xs_ref[...] = (
        jnp.dot(x, w1_ref[0:D, :], preferred_element_type=jnp.float32)
        + b1_ref[...]
    )
    xr_ref[...] = jnp.dot(x, w1_ref[D:2 * D, :], preferred_element_type=jnp.float32)


def _edge_body(s_ref, e_ref, w1e_ref, w2_ref, b2_ref, ne_ref, eo_ref):
    e = e_ref[...]
    pe = jnp.dot(e, w1e_ref[...], preferred_element_type=jnp.float32)
    h = jnp.maximum(s_ref[...] + pe, 0.0)
    tmp = jnp.dot(h, w2_ref[...], preferred_element_type=jnp.float32) + b2_ref[...]
    ne_ref[...] = tmp
    eo_ref[...] = tmp + e


def _edge_body_alias(s_ref, e_ref, w1e_ref, w2_ref, b2_ref, eo_in_ref,
                     ne_ref, eo_ref):
    del eo_in_ref
    _edge_body(s_ref, e_ref, w1e_ref, w2_ref, b2_ref, ne_ref, eo_ref)


def _node_body(x_ref, p0_ref, p1_ref, w1_ref, b1_ref, w2_ref, b2_ref, out_ref):
    x = x_ref[...]
    agg = (p0_ref[0] + p0_ref[1]) + (p1_ref[0] + p1_ref[1])
    h = jnp.maximum(
        jnp.dot(x, w1_ref[0:D, :], preferred_element_type=jnp.float32)
        + jnp.dot(agg, w1_ref[D:2 * D, :], preferred_element_type=jnp.float32)
        + b1_ref[...],
        0.0,
    )
    out_ref[...] = (
        jnp.dot(h, w2_ref[...], preferred_element_type=jnp.float32)
        + b2_ref[...]
        + x
    )


def _make_gather(soff):
    @functools.partial(
        pl.kernel,
        mesh=_mesh,
        out_type=jax.ShapeDtypeStruct((ES, D), jnp.float32),
        scratch_types=(
            [pltpu.VMEM((GCHUNK,), jnp.int32) for _ in range(2 * NRING)]
            + [pltpu.VMEM((GCHUNK, D), jnp.float32) for _ in range(2 * NRING)]
            + [pltpu.SemaphoreType.DMA for _ in range(3 * NRING)]
        ),
    )
    def gather_sc(xs_hbm, xr_hbm, snd_hbm, rcv_hbm, s_hbm, *scr):
        idx_s = scr[0:NRING]
        idx_r = scr[NRING:2 * NRING]
        rows_s = scr[2 * NRING:3 * NRING]
        rows_r = scr[3 * NRING:4 * NRING]
        sem_i = scr[4 * NRING:5 * NRING]
        sem_g = scr[5 * NRING:6 * NRING]
        sem_w = scr[6 * NRING:7 * NRING]

        wid = lax.axis_index("s") * NC + lax.axis_index("c")
        base = wid * EPW
        ibase = soff + base

        def _drain_wb(b):
            pltpu.make_async_copy(
                rows_s[b], s_hbm.at[pl.ds(base, GCHUNK)], sem_w[b]).wait()

        def _front(b):
            pltpu.make_async_copy(
                snd_hbm.at[pl.ds(ibase, GCHUNK)], idx_s[b], sem_i[b]).wait()
            pltpu.make_async_copy(
                rcv_hbm.at[pl.ds(ibase, GCHUNK)], idx_r[b], sem_i[b]).wait()
            pltpu.async_copy(xs_hbm.at[idx_s[b]], rows_s[b], sem_g[b])
            pltpu.async_copy(xr_hbm.at[idx_r[b]], rows_r[b], sem_g[b])

        def _back(j, b, prefetch):
            off = base + j * GCHUNK
            pltpu.make_async_copy(
                xs_hbm.at[idx_s[b]], rows_s[b], sem_g[b]).wait()
            pltpu.make_async_copy(
                xr_hbm.at[idx_r[b]], rows_r[b], sem_g[b]).wait()

            def vadd(r, carry):
                for q in range(D // 16):
                    sl = pl.ds(q * 16, 16)
                    rows_s[b][r, sl] = rows_s[b][r, sl] + rows_r[b][r, sl]
                return carry

            lax.fori_loop(0, GCHUNK, vadd, 0)
            pltpu.async_copy(rows_s[b], s_hbm.at[pl.ds(off, GCHUNK)], sem_w[b])
            if prefetch:
                @pl.when(j + NRING < GNCHUNK)
                def _prefetch():
                    ioff = ibase + (j + NRING) * GCHUNK
                    pltpu.async_copy(
                        snd_hbm.at[pl.ds(ioff, GCHUNK)], idx_s[b], sem_i[b])
                    pltpu.async_copy(
                        rcv_hbm.at[pl.ds(ioff, GCHUNK)], idx_r[b], sem_i[b])

        for b in range(NRING):
            ioff = ibase + b * GCHUNK
            pltpu.async_copy(snd_hbm.at[pl.ds(ioff, GCHUNK)], idx_s[b], sem_i[b])
            pltpu.async_copy(rcv_hbm.at[pl.ds(ioff, GCHUNK)], idx_r[b], sem_i[b])

        def outer(g, carry):
            for b in range(NRING):
                k = g * NRING + b
                j = k - GOFF
                bj = (b + NRING - GOFF) % NRING

                @pl.when(g > 0)
                def _drain():
                    _drain_wb(b)

                _front(b)

                @pl.when(j >= 0)
                def _backstage():
                    _back(j, bj, prefetch=True)
            return carry

        lax.fori_loop(0, NGOUT, outer, 0)

        k0 = NGOUT * NRING
        for b in range(GTAIL):
            _drain_wb(b)
            _front(b)
            _back(k0 - GOFF + b, (b + NRING - GOFF) % NRING, prefetch=False)
        for b in range(GOFF):
            jj = GNCHUNK - GOFF + b
            _back(jj, jj % NRING, prefetch=False)
        for b in range(NRING):
            _drain_wb(b)

    return gather_sc


def _make_scatter(soff):
    @functools.partial(
        pl.kernel,
        mesh=_mesh,
        out_type=jax.ShapeDtypeStruct((NC, NP, D), jnp.float32),
        scratch_types=(
            [pltpu.VMEM((SCHUNK,), jnp.int32) for _ in range(NBUF)]
            + [pltpu.VMEM((SCHUNK, D), jnp.float32) for _ in range(NBUF)]
            + [pltpu.VMEM_SHARED((NP, D), jnp.float32)]
            + [pltpu.SemaphoreType.DMA for _ in range(NBUF)]
        ),
    )
    def scatter_sc(ne_hbm, rcv_hbm, zero_hbm, parts_hbm, *scr):
        idx = scr[0:NBUF]
        chunk = scr[NBUF:2 * NBUF]
        acc = scr[2 * NBUF]
        sem_i = scr[2 * NBUF + 1:3 * NBUF + 1]

        cid = lax.axis_index("c")
        sid = lax.axis_index("s")
        wid = sid * NC + cid

        pltpu.sync_copy(zero_hbm.at[pl.ds(0, RSTEP)], chunk[0])

        def zbody(i, carry):
            r0 = sid * RPS + i * RSTEP
            pltpu.sync_copy(chunk[0], acc.at[pl.ds(r0, RSTEP)])
            return carry

        lax.fori_loop(0, RPS // RSTEP, zbody, 0)
        plsc.subcore_barrier()

        base = wid * EPW
        ibase = soff + base

        for b in range(NBUF):
            pltpu.async_copy(
                rcv_hbm.at[pl.ds(ibase + b * SCHUNK, SCHUNK)], idx[b], sem_i[b])
            pltpu.async_copy(
                ne_hbm.at[pl.ds(base + b * SCHUNK, SCHUNK)], chunk[b], sem_i[b])

        def outer(g, carry):
            for b in range(NBUF):
                k = g * NBUF + b
                pltpu.make_async_copy(
                    rcv_hbm.at[pl.ds(ibase, SCHUNK)], idx[b], sem_i[b]).wait()
                pltpu.make_async_copy(
                    ne_hbm.at[pl.ds(base, SCHUNK)], chunk[b], sem_i[b]).wait()
                pltpu.sync_copy(chunk[b], acc.at[idx[b]], add=True)

                @pl.when(k + NBUF < SNCHUNK)
                def _prefetch():
                    k2 = k + NBUF
                    pltpu.async_copy(
                        rcv_hbm.at[pl.ds(ibase + k2 * SCHUNK, SCHUNK)],
                        idx[b], sem_i[b])
                    pltpu.async_copy(
                        ne_hbm.at[pl.ds(base + k2 * SCHUNK, SCHUNK)],
                        chunk[b], sem_i[b])
            return carry

        lax.fori_loop(0, SNOUTER, outer, 0)
        plsc.subcore_barrier()

        def obody(i, carry):
            r0 = sid * RPS + i * RSTEP
            pltpu.sync_copy(acc.at[pl.ds(r0, RSTEP)], chunk[0])
            pltpu.sync_copy(chunk[0], parts_hbm.at[cid, pl.ds(r0, RSTEP)])
            return carry

        lax.fori_loop(0, RPS // RSTEP, obody, 0)

    return scatter_sc


_gather_calls = [_make_gather(s * ES) for s in range(NSPLIT)]
_scatter_calls = [_make_scatter(s * ES) for s in range(NSPLIT)]


def kernel(node_features, edge_features, me_w1, me_b1, me_w2, me_b2,
           nm_w1, nm_b1, nm_w2, nm_b2, senders, receivers):
    snd = senders.astype(jnp.int32)
    rcv = receivers.astype(jnp.int32)

    BN = 1000
    xs, xr = pl.pallas_call(
        _proj_body,
        grid=(N // BN,),
        in_specs=[
            pl.BlockSpec((BN, D), lambda i: (i, 0)),
            pl.BlockSpec((3 * D, D), lambda i: (0, 0)),
            pl.BlockSpec((1, D), lambda i: (0, 0)),
        ],
        out_specs=[pl.BlockSpec((BN, D), lambda i: (i, 0))] * 2,
        out_shape=[jax.ShapeDtypeStruct((N, D), jnp.float32)] * 2,
    )(node_features, me_w1, me_b1.reshape(1, D))

    svals = [g(xs, xr, snd, rcv) for g in _gather_calls]

    BE = 4000
    EBS = ES // BE
    w1e = me_w1[2 * D:3 * D]
    b2 = me_b2.reshape(1, D)

    def _edge_specs(s):
        return [
            pl.BlockSpec((BE, D), lambda i: (i, 0)),
            pl.BlockSpec((BE, D), lambda i, s=s: (s * EBS + i, 0)),
            pl.BlockSpec((D, D), lambda i: (0, 0)),
            pl.BlockSpec((D, D), lambda i: (0, 0)),
            pl.BlockSpec((1, D), lambda i: (0, 0)),
        ]

    ne0, eo = pl.pallas_call(
        _edge_body,
        grid=(EBS,),
        in_specs=_edge_specs(0),
        out_specs=[
            pl.BlockSpec((BE, D), lambda i: (i, 0)),
            pl.BlockSpec((BE, D), lambda i: (i, 0)),
        ],
        out_shape=[
            jax.ShapeDtypeStruct((ES, D), jnp.float32),
            jax.ShapeDtypeStruct((E, D), jnp.float32),
        ],
    )(svals[0], edge_features, w1e, me_w2, b2)

    ne1, eo = pl.pallas_call(
        _edge_body_alias,
        grid=(EBS,),
        in_specs=_edge_specs(1) + [pl.BlockSpec(memory_space=pltpu.HBM)],
        out_specs=[
            pl.BlockSpec((BE, D), lambda i: (i, 0)),
            pl.BlockSpec((BE, D), lambda i: (EBS + i, 0)),
        ],
        out_shape=[
            jax.ShapeDtypeStruct((ES, D), jnp.float32),
            jax.ShapeDtypeStruct((E, D), jnp.float32),
        ],
        input_output_aliases={5: 1},
    )(svals[1], edge_features, w1e, me_w2, b2, eo)

    zeros = jnp.zeros((NP, D), jnp.float32)
    parts0 = _scatter_calls[0](ne0, rcv, zeros)
    parts1 = _scatter_calls[1](ne1, rcv, zeros)

    node_out = pl.pallas_call(
        _node_body,
        grid=(N // BN,),
        in_specs=[
            pl.BlockSpec((BN, D), lambda i: (i, 0)),
            pl.BlockSpec((NC, BN, D), lambda i: (0, i, 0)),
            pl.BlockSpec((NC, BN, D), lambda i: (0, i, 0)),
            pl.BlockSpec((2 * D, D), lambda i: (0, 0)),
            pl.BlockSpec((1, D), lambda i: (0, 0)),
            pl.BlockSpec((D, D), lambda i: (0, 0)),
            pl.BlockSpec((1, D), lambda i: (0, 0)),
        ],
        out_specs=pl.BlockSpec((BN, D), lambda i: (i, 0)),
        out_shape=jax.ShapeDtypeStruct((N, D), jnp.float32),
    )(node_features, parts0, parts1, nm_w1, nm_b1.reshape(1, D), nm_w2,
      nm_b2.reshape(1, D))

    return node_out, eo


# speedup vs baseline: 1.3837x; 1.0084x over previous
import functools

import jax
import jax.numpy as jnp
from jax import lax
from jax.experimental import pallas as pl
from jax.experimental.pallas import tpu as pltpu
from jax.experimental.pallas import tpu_sc as plsc

N = 10000
E = 320000
D = 128

NSPLIT = 2
ES_LIST = (192000, 128000)
SOFF_LIST = (0, 192000)
NC = 2
NS = 16
NW = NC * NS

GCHUNK = 40
NRING = 10
GOFF = 5

NBUF = 5
SCHUNK = 40
NP = 10240
RPS = NP // NS
RSTEP = SCHUNK

_mesh = plsc.VectorSubcoreMesh(core_axis_name="c", subcore_axis_name="s")


def _proj_body(x_ref, w1_ref, b1_ref, xs_ref, xr_ref):
    x = x_ref[...]
    xs_ref[...] = (
        jnp.dot(x, w1_ref[0:D, :], preferred_element_type=jnp.float32)
        + b1_ref[...]
    )
    xr_ref[...] = jnp.dot(x, w1_ref[D:2 * D, :], preferred_element_type=jnp.float32)


def _edge_body(s_ref, e_ref, w1e_ref, w2_ref, b2_ref, ne_ref, eo_ref):
    e = e_ref[...]
    pe = jnp.dot(e, w1e_ref[...], preferred_element_type=jnp.float32)
    h = jnp.maximum(s_ref[...] + pe, 0.0)
    tmp = jnp.dot(h, w2_ref[...], preferred_element_type=jnp.float32) + b2_ref[...]
    ne_ref[...] = tmp
    eo_ref[...] = tmp + e


def _edge_body_alias(s_ref, e_ref, w1e_ref, w2_ref, b2_ref, eo_in_ref,
                     ne_ref, eo_ref):
    del eo_in_ref
    _edge_body(s_ref, e_ref, w1e_ref, w2_ref, b2_ref, ne_ref, eo_ref)


def _node_body(x_ref, p0_ref, p1_ref, w1_ref, b1_ref, w2_ref, b2_ref, out_ref):
    x = x_ref[...]
    agg = (p0_ref[0] + p0_ref[1]) + (p1_ref[0] + p1_ref[1])
    h = jnp.maximum(
        jnp.dot(x, w1_ref[0:D, :], preferred_element_type=jnp.float32)
        + jnp.dot(agg, w1_ref[D:2 * D, :], preferred_element_type=jnp.float32)
        + b1_ref[...],
        0.0,
    )
    out_ref[...] = (
        jnp.dot(h, w2_ref[...], preferred_element_type=jnp.float32)
        + b2_ref[...]
        + x
    )


def _make_gather(soff, es):
    epw = es // NW
    gnchunk = epw // GCHUNK
    ngout = gnchunk // NRING
    gtail = gnchunk - ngout * NRING

    @functools.partial(
        pl.kernel,
        mesh=_mesh,
        out_type=jax.ShapeDtypeStruct((es, D), jnp.float32),
        scratch_types=(
            [pltpu.VMEM((GCHUNK,), jnp.int32) for _ in range(2 * NRING)]
            + [pltpu.VMEM((GCHUNK, D), jnp.float32) for _ in range(2 * NRING)]
            + [pltpu.SemaphoreType.DMA for _ in range(3 * NRING)]
        ),
    )
    def gather_sc(xs_hbm, xr_hbm, snd_hbm, rcv_hbm, s_hbm, *scr):
        idx_s = scr[0:NRING]
        idx_r = scr[NRING:2 * NRING]
        rows_s = scr[2 * NRING:3 * NRING]
        rows_r = scr[3 * NRING:4 * NRING]
        sem_i = scr[4 * NRING:5 * NRING]
        sem_g = scr[5 * NRING:6 * NRING]
        sem_w = scr[6 * NRING:7 * NRING]

        wid = lax.axis_index("s") * NC + lax.axis_index("c")
        base = wid * epw
        ibase = soff + base

        def _drain_wb(b):
            pltpu.make_async_copy(
                rows_s[b], s_hbm.at[pl.ds(base, GCHUNK)], sem_w[b]).wait()

        def _front(b):
            pltpu.make_async_copy(
                snd_hbm.at[pl.ds(ibase, GCHUNK)], idx_s[b], sem_i[b]).wait()
            pltpu.make_async_copy(
                rcv_hbm.at[pl.ds(ibase, GCHUNK)], idx_r[b], sem_i[b]).wait()
            pltpu.async_copy(xs_hbm.at[idx_s[b]], rows_s[b], sem_g[b])
            pltpu.async_copy(xr_hbm.at[idx_r[b]], rows_r[b], sem_g[b])

        def _back(j, b, prefetch):
            off = base + j * GCHUNK
            pltpu.make_async_copy(
                xs_hbm.at[idx_s[b]], rows_s[b], sem_g[b]).wait()
            pltpu.make_async_copy(
                xr_hbm.at[idx_r[b]], rows_r[b], sem_g[b]).wait()

            def vadd(r, carry):
                for q in range(D // 16):
                    sl = pl.ds(q * 16, 16)
                    rows_s[b][r, sl] = rows_s[b][r, sl] + rows_r[b][r, sl]
                return carry

            lax.fori_loop(0, GCHUNK, vadd, 0)
            pltpu.async_copy(rows_s[b], s_hbm.at[pl.ds(off, GCHUNK)], sem_w[b])
            if prefetch:
                @pl.when(j + NRING < gnchunk)
                def _prefetch():
                    ioff = ibase + (j + NRING) * GCHUNK
                    pltpu.async_copy(
                        snd_hbm.at[pl.ds(ioff, GCHUNK)], idx_s[b], sem_i[b])
                    pltpu.async_copy(
                        rcv_hbm.at[pl.ds(ioff, GCHUNK)], idx_r[b], sem_i[b])

        for b in range(NRING):
            ioff = ibase + b * GCHUNK
            pltpu.async_copy(snd_hbm.at[pl.ds(ioff, GCHUNK)], idx_s[b], sem_i[b])
            pltpu.async_copy(rcv_hbm.at[pl.ds(ioff, GCHUNK)], idx_r[b], sem_i[b])

        def outer(g, carry):
            for b in range(NRING):
                k = g * NRING + b
                j = k - GOFF
                bj = (b + NRING - GOFF) % NRING

                @pl.when(g > 0)
                def _drain():
                    _drain_wb(b)

                _front(b)

                @pl.when(j >= 0)
                def _backstage():
                    _back(j, bj, prefetch=True)
            return carry

        lax.fori_loop(0, ngout, outer, 0)

        k0 = ngout * NRING
        for b in range(gtail):
            _drain_wb(b)
            _front(b)
            _back(k0 - GOFF + b, (b + NRING - GOFF) % NRING, prefetch=False)
        for b in range(GOFF):
            jj = gnchunk - GOFF + b
            _back(jj, jj % NRING, prefetch=False)
        for b in range(NRING):
            _drain_wb(b)

    return gather_sc


def _make_scatter(soff, es):
    epw = es // NW
    snchunk = epw // SCHUNK
    snouter = snchunk // NBUF

    @functools.partial(
        pl.kernel,
        mesh=_mesh,
        out_type=jax.ShapeDtypeStruct((NC, NP, D), jnp.float32),
        scratch_types=(
            [pltpu.VMEM((SCHUNK,), jnp.int32) for _ in range(NBUF)]
            + [pltpu.VMEM((SCHUNK, D), jnp.float32) for _ in range(NBUF)]
            + [pltpu.VMEM_SHARED((NP, D), jnp.float32)]
            + [pltpu.SemaphoreType.DMA for _ in range(NBUF)]
        ),
    )
    def scatter_sc(ne_hbm, rcv_hbm, zero_hbm, parts_hbm, *scr):
        idx = scr[0:NBUF]
        chunk = scr[NBUF:2 * NBUF]
        acc = scr[2 * NBUF]
        sem_i = scr[2 * NBUF + 1:3 * NBUF + 1]

        cid = lax.axis_index("c")
        sid = lax.axis_index("s")
        wid = sid * NC + cid

        pltpu.sync_copy(zero_hbm.at[pl.ds(0, RSTEP)], chunk[0])

        def zbody(i, carry):
            r0 = sid * RPS + i * RSTEP
            pltpu.sync_copy(chunk[0], acc.at[pl.ds(r0, RSTEP)])
            return carry

        lax.fori_loop(0, RPS // RSTEP, zbody, 0)
        plsc.subcore_barrier()

        base = wid * epw
        ibase = soff + base

        for b in range(NBUF):
            pltpu.async_copy(
                rcv_hbm.at[pl.ds(ibase + b * SCHUNK, SCHUNK)], idx[b], sem_i[b])
            pltpu.async_copy(
                ne_hbm.at[pl.ds(base + b * SCHUNK, SCHUNK)], chunk[b], sem_i[b])

        def outer(g, carry):
            for b in range(NBUF):
                k = g * NBUF + b
                pltpu.make_async_copy(
                    rcv_hbm.at[pl.ds(ibase, SCHUNK)], idx[b], sem_i[b]).wait()
                pltpu.make_async_copy(
                    ne_hbm.at[pl.ds(base, SCHUNK)], chunk[b], sem_i[b]).wait()
                pltpu.sync_copy(chunk[b], acc.at[idx[b]], add=True)

                @pl.when(k + NBUF < snchunk)
                def _prefetch():
                    k2 = k + NBUF
                    pltpu.async_copy(
                        rcv_hbm.at[pl.ds(ibase + k2 * SCHUNK, SCHUNK)],
                        idx[b], sem_i[b])
                    pltpu.async_copy(
                        ne_hbm.at[pl.ds(base + k2 * SCHUNK, SCHUNK)],
                        chunk[b], sem_i[b])
            return carry

        lax.fori_loop(0, snouter, outer, 0)
        plsc.subcore_barrier()

        def obody(i, carry):
            r0 = sid * RPS + i * RSTEP
            pltpu.sync_copy(acc.at[pl.ds(r0, RSTEP)], chunk[0])
            pltpu.sync_copy(chunk[0], parts_hbm.at[cid, pl.ds(r0, RSTEP)])
            return carry

        lax.fori_loop(0, RPS // RSTEP, obody, 0)

    return scatter_sc


_gather_calls = [_make_gather(SOFF_LIST[s], ES_LIST[s]) for s in range(NSPLIT)]
_scatter_calls = [_make_scatter(SOFF_LIST[s], ES_LIST[s]) for s in range(NSPLIT)]


def kernel(node_features, edge_features, me_w1, me_b1, me_w2, me_b2,
           nm_w1, nm_b1, nm_w2, nm_b2, senders, receivers):
    snd = senders.astype(jnp.int32)
    rcv = receivers.astype(jnp.int32)

    BN = 1000
    xs, xr = pl.pallas_call(
        _proj_body,
        grid=(N // BN,),
        in_specs=[
            pl.BlockSpec((BN, D), lambda i: (i, 0)),
            pl.BlockSpec((3 * D, D), lambda i: (0, 0)),
            pl.BlockSpec((1, D), lambda i: (0, 0)),
        ],
        out_specs=[pl.BlockSpec((BN, D), lambda i: (i, 0))] * 2,
        out_shape=[jax.ShapeDtypeStruct((N, D), jnp.float32)] * 2,
    )(node_features, me_w1, me_b1.reshape(1, D))

    svals = [g(xs, xr, snd, rcv) for g in _gather_calls]

    BE = 4000
    EBS0 = ES_LIST[0] // BE
    EBS1 = ES_LIST[1] // BE
    w1e = me_w1[2 * D:3 * D]
    b2 = me_b2.reshape(1, D)

    def _edge_specs(blk_off):
        return [
            pl.BlockSpec((BE, D), lambda i: (i, 0)),
            pl.BlockSpec((BE, D), lambda i: (blk_off + i, 0)),
            pl.BlockSpec((D, D), lambda i: (0, 0)),
            pl.BlockSpec((D, D), lambda i: (0, 0)),
            pl.BlockSpec((1, D), lambda i: (0, 0)),
        ]

    ne0, eo = pl.pallas_call(
        _edge_body,
        grid=(EBS0,),
        in_specs=_edge_specs(0),
        out_specs=[
            pl.BlockSpec((BE, D), lambda i: (i, 0)),
            pl.BlockSpec((BE, D), lambda i: (i, 0)),
        ],
        out_shape=[
            jax.ShapeDtypeStruct((ES_LIST[0], D), jnp.float32),
            jax.ShapeDtypeStruct((E, D), jnp.float32),
        ],
    )(svals[0], edge_features, w1e, me_w2, b2)

    ne1, eo = pl.pallas_call(
        _edge_body_alias,
        grid=(EBS1,),
        in_specs=_edge_specs(EBS0) + [pl.BlockSpec(memory_space=pltpu.HBM)],
        out_specs=[
            pl.BlockSpec((BE, D), lambda i: (i, 0)),
            pl.BlockSpec((BE, D), lambda i: (EBS0 + i, 0)),
        ],
        out_shape=[
            jax.ShapeDtypeStruct((ES_LIST[1], D), jnp.float32),
            jax.ShapeDtypeStruct((E, D), jnp.float32),
        ],
        input_output_aliases={5: 1},
    )(svals[1], edge_features, w1e, me_w2, b2, eo)

    zeros = jnp.zeros((NP, D), jnp.float32)
    parts0 = _scatter_calls[0](ne0, rcv, zeros)
    parts1 = _scatter_calls[1](ne1, rcv, zeros)

    node_out = pl.pallas_call(
        _node_body,
        grid=(N // BN,),
        in_specs=[
            pl.BlockSpec((BN, D), lambda i: (i, 0)),
            pl.BlockSpec((NC, BN, D), lambda i: (0, i, 0)),
            pl.BlockSpec((NC, BN, D), lambda i: (0, i, 0)),
            pl.BlockSpec((2 * D, D), lambda i: (0, 0)),
            pl.BlockSpec((1, D), lambda i: (0, 0)),
            pl.BlockSpec((D, D), lambda i: (0, 0)),
            pl.BlockSpec((1, D), lambda i: (0, 0)),
        ],
        out_specs=pl.BlockSpec((BN, D), lambda i: (i, 0)),
        out_shape=jax.ShapeDtypeStruct((N, D), jnp.float32),
    )(node_features, parts0, parts1, nm_w1, nm_b1.reshape(1, D), nm_w2,
      nm_b2.reshape(1, D))

    return node_out, eo
